# Initial kernel scaffold; baseline (speedup 1.0000x reference)
import jax
import jax.numpy as jnp
from jax.experimental import pallas as pl


def kernel(nfeats, edge_index, W1, b1, W2, b2, W3, b3, Wc, bc):
    raise NotImplementedError("write your pallas kernel here")


# speedup vs baseline: 2.8675x; 2.8675x over previous
import functools

import jax
import jax.numpy as jnp
from jax import lax
from jax.experimental import pallas as pl
from jax.experimental.pallas import tpu as pltpu
from jax.experimental.pallas import tpu_sc as plsc

N = 10000
E = 320000
D_IN = 128
D_HID = 256

NC = 2
NS = 16
LANE = 128

K_ROWS = -(-E // (NS * LANE))
EPAD = NS * K_ROWS * LANE

NH = 5200
NBH = 13
PR = 5248
SLICE = PR // NS
NDUM = 48

DROWS = 10112
DSLICE = DROWS // NS

_MESH = plsc.VectorSubcoreMesh(core_axis_name="c", subcore_axis_name="s",
                               num_cores=NC, num_subcores=NS)


def _zero_vmem_2d(ref, nrows, ncols):
    zeros16 = jnp.zeros((16,), jnp.float32)
    ncol_chunks = ncols // 16

    def body(r, _):
        for cchunk in range(ncol_chunks):
            ref[r, pl.ds(cchunk * 16, 16)] = zeros16
        return _

    lax.fori_loop(0, nrows, body, None)


@functools.partial(
    pl.kernel,
    out_type=jax.ShapeDtypeStruct((NC * DROWS,), jnp.float32),
    mesh=_MESH,
    scratch_types=[
        pltpu.VMEM((K_ROWS, LANE), jnp.int32),
        pltpu.VMEM((LANE,), jnp.float32),
        pltpu.VMEM((LANE,), jnp.float32),
        pltpu.VMEM((DSLICE,), jnp.float32),
        pltpu.VMEM_SHARED((DROWS,), jnp.float32),
    ],
)
def _sc_degrees(idx_hbm, out_hbm, idx_v, ones_v, zero_v, wb_v, hist_sp):
    c = lax.axis_index("c")
    s = lax.axis_index("s")
    for i in range(LANE // 16):
        ones_v[pl.ds(i * 16, 16)] = jnp.ones((16,), jnp.float32)
        zero_v[pl.ds(i * 16, 16)] = jnp.zeros((16,), jnp.float32)
    base = s * DSLICE
    for k in range(4):
        pltpu.sync_copy(zero_v, hist_sp.at[pl.ds(base + k * LANE, LANE)])
    pltpu.sync_copy(zero_v.at[pl.ds(0, 120)],
                    hist_sp.at[pl.ds(base + 4 * LANE, 120)])
    plsc.subcore_barrier()

    pltpu.sync_copy(idx_hbm.at[c, s], idx_v)

    def body(j, _):
        pltpu.sync_copy(ones_v, hist_sp.at[idx_v.at[j]], add=True)
        return _

    lax.fori_loop(0, K_ROWS, body, None)
    plsc.subcore_barrier()
    pltpu.sync_copy(hist_sp.at[pl.ds(base, DSLICE)], wb_v)
    pltpu.sync_copy(wb_v, out_hbm.at[pl.ds(c * DROWS + base, DSLICE)])


@functools.partial(
    pl.kernel,
    out_type=jax.ShapeDtypeStruct((NC, 2, PR, LANE), jnp.float32),
    mesh=_MESH,
    scratch_types=[
        pltpu.VMEM((K_ROWS, LANE), jnp.int32),
        pltpu.VMEM((K_ROWS, LANE), jnp.int32),
        pltpu.VMEM((LANE, LANE), jnp.float32),
        pltpu.VMEM((LANE, LANE), jnp.float32),
        pltpu.VMEM_SHARED((PR, LANE), jnp.float32),
        pltpu.SemaphoreType.DMA,
    ],
)
def _sc_aggregate(h_hbm, src_hbm, dst_hbm, out_hbm,
                  src_v, dst_v, rows_v, zero_v, agg_sp, sem):
    c = lax.axis_index("c")
    s = lax.axis_index("s")
    _zero_vmem_2d(zero_v, LANE, LANE)
    base = s * SLICE
    pltpu.sync_copy(src_hbm.at[c, s], src_v)

    for r in range(2):
        for k in range(2):
            pltpu.sync_copy(zero_v, agg_sp.at[pl.ds(base + k * LANE, LANE)])
        pltpu.sync_copy(zero_v.at[pl.ds(0, 72)],
                        agg_sp.at[pl.ds(base + 2 * LANE, 72)])
        plsc.subcore_barrier()

        pltpu.sync_copy(dst_hbm.at[r, s], dst_v)

        def body(j, _):
            pltpu.async_copy(h_hbm.at[src_v.at[j]], rows_v, sem).wait()
            pltpu.sync_copy(rows_v, agg_sp.at[dst_v.at[j]], add=True)
            return _

        lax.fori_loop(0, K_ROWS, body, None)
        plsc.subcore_barrier()
        for k in range(2):
            pltpu.sync_copy(agg_sp.at[pl.ds(base + k * LANE, LANE)], rows_v)
            pltpu.sync_copy(rows_v,
                            out_hbm.at[c, r, pl.ds(base + k * LANE, LANE)])
        pltpu.sync_copy(agg_sp.at[pl.ds(base + 2 * LANE, 72)],
                        rows_v.at[pl.ds(0, 72)])
        pltpu.sync_copy(rows_v.at[pl.ds(0, 72)],
                        out_hbm.at[c, r, pl.ds(base + 2 * LANE, 72)])


_R = 400
_NB = N // _R


def _agg_index(i):
    r = (i >= NBH).astype(jnp.int32)
    return r, i - r * NBH


def _t1_body(x_ref, w_ref, dego_ref, out_ref):
    ns = lax.rsqrt(jnp.maximum(dego_ref[...], 1.0))
    z = jnp.dot(x_ref[...], w_ref[0], preferred_element_type=jnp.float32)
    out_ref[0] = z * ns


def _t1(x, w1, deg_out):
    return pl.pallas_call(
        _t1_body,
        grid=(_NB, NC),
        in_specs=[
            pl.BlockSpec((_R, D_IN), lambda i, k: (i, 0)),
            pl.BlockSpec((1, D_IN, LANE), lambda i, k: (k, 0, 0)),
            pl.BlockSpec((_R, 1), lambda i, k: (i, 0)),
        ],
        out_specs=pl.BlockSpec((1, _R, LANE), lambda i, k: (k, i, 0)),
        out_shape=jax.ShapeDtypeStruct((NC, N, LANE), jnp.float32),
    )(x, w1, deg_out)


def _tmid_body(agg_ref, degi_ref, dego_ref, b_ref, w_ref, out_ref):
    nd = lax.rsqrt(jnp.maximum(degi_ref[...], 1.0))
    x = jnp.concatenate([agg_ref[0, 0], agg_ref[1, 0]], axis=1)
    x = jax.nn.relu(x * nd + b_ref[...])
    z = jnp.dot(x, w_ref[0], preferred_element_type=jnp.float32)
    ns = lax.rsqrt(jnp.maximum(dego_ref[...], 1.0))
    out_ref[0] = z * ns


def _tmid(agg, deg_in, deg_out, b, w):
    def agg_map(i, k):
        r, ib = _agg_index(i)
        return (0, r, ib, 0)

    return pl.pallas_call(
        _tmid_body,
        grid=(_NB, NC),
        in_specs=[
            pl.BlockSpec((NC, 1, _R, LANE), agg_map),
            pl.BlockSpec((_R, 1), lambda i, k: (i, 0)),
            pl.BlockSpec((_R, 1), lambda i, k: (i, 0)),
            pl.BlockSpec((1, D_HID), lambda i, k: (0, 0)),
            pl.BlockSpec((1, D_HID, LANE), lambda i, k: (k, 0, 0)),
        ],
        out_specs=pl.BlockSpec((1, _R, LANE), lambda i, k: (k, i, 0)),
        out_shape=jax.ShapeDtypeStruct((NC, N, LANE), jnp.float32),
    )(agg, deg_in, deg_out, b, w)


def _t4_body(agg_ref, degi_ref, b_ref, wc_ref, bc_ref, out_ref, acc_ref):
    i = pl.program_id(0)
    nd = lax.rsqrt(jnp.maximum(degi_ref[...], 1.0))
    x = jnp.concatenate([agg_ref[0, 0], agg_ref[1, 0]], axis=1)
    x = jax.nn.relu(x * nd + b_ref[...])
    part = jnp.sum(x, axis=0, keepdims=True)

    @pl.when(i == 0)
    def _():
        acc_ref[...] = part

    @pl.when(i > 0)
    def _():
        acc_ref[...] = acc_ref[...] + part

    @pl.when(i == _NB - 1)
    def _():
        hg = acc_ref[...] * (1.0 / N)
        logit = jnp.dot(hg, wc_ref[...],
                        preferred_element_type=jnp.float32) + bc_ref[...]
        out_ref[...] = jax.nn.sigmoid(logit)


def _t4(agg, deg_in, b, wc, bc):
    def agg_map(i):
        r, ib = _agg_index(i)
        return (0, r, ib, 0)

    return pl.pallas_call(
        _t4_body,
        grid=(_NB,),
        in_specs=[
            pl.BlockSpec((NC, 1, _R, LANE), agg_map),
            pl.BlockSpec((_R, 1), lambda i: (i, 0)),
            pl.BlockSpec((1, D_HID), lambda i: (0, 0)),
            pl.BlockSpec((D_HID, 1), lambda i: (0, 0)),
            pl.BlockSpec((1, 1), lambda i: (0, 0)),
        ],
        out_specs=pl.BlockSpec((1, 1), lambda i: (0, 0)),
        out_shape=jax.ShapeDtypeStruct((1, 1), jnp.float32),
        scratch_shapes=[pltpu.VMEM((1, D_HID), jnp.float32)],
        compiler_params=pltpu.CompilerParams(
            dimension_semantics=("arbitrary",)),
    )(agg, deg_in, b, wc, bc)


def kernel(nfeats, edge_index, W1, b1, W2, b2, W3, b3, Wc, bc):
    src = edge_index[0].astype(jnp.int32)
    dst = edge_index[1].astype(jnp.int32)
    pad = EPAD - E
    spread_pad = jnp.arange(pad, dtype=jnp.int32) % NDUM
    src_p0 = jnp.concatenate([src, jnp.zeros((pad,), jnp.int32)])
    src_pd = jnp.concatenate([src, N + spread_pad])
    dst_pd = jnp.concatenate([dst, N + spread_pad])

    deg_idx = jnp.stack([src_pd, dst_pd]).reshape(NC, NS, K_ROWS, LANE)
    agg_src = jnp.stack([src_p0, src_p0 + N]).reshape(NC, NS, K_ROWS, LANE)

    spread = jnp.arange(EPAD, dtype=jnp.int32) % NDUM
    dst0 = jnp.where(dst_pd < NH, dst_pd, NH + spread)
    dst1 = jnp.where(dst_pd >= NH, dst_pd - NH, N - NH + spread)
    agg_dst = jnp.stack([dst0, dst1]).reshape(2, NS, K_ROWS, LANE)

    degs = _sc_degrees(deg_idx).reshape(NC, DROWS)
    deg_out = degs[0, :N].reshape(N, 1)
    deg_in = degs[1, :N].reshape(N, 1)

    w1q = W1.reshape(D_IN, NC, LANE).transpose(1, 0, 2)
    w2q = W2.reshape(D_HID, NC, LANE).transpose(1, 0, 2)
    w3q = W3.reshape(D_HID, NC, LANE).transpose(1, 0, 2)
    b1r = b1.reshape(1, D_HID)
    b2r = b2.reshape(1, D_HID)
    b3r = b3.reshape(1, D_HID)
    bcr = bc.reshape(1, 1)

    h1 = _t1(nfeats, w1q, deg_out)
    agg1 = _sc_aggregate(h1.reshape(NC * N, LANE), agg_src, agg_dst)
    h2 = _tmid(agg1, deg_in, deg_out, b1r, w2q)
    agg2 = _sc_aggregate(h2.reshape(NC * N, LANE), agg_src, agg_dst)
    h3 = _tmid(agg2, deg_in, deg_out, b2r, w3q)
    agg3 = _sc_aggregate(h3.reshape(NC * N, LANE), agg_src, agg_dst)
    return _t4(agg3, deg_in, b3r, Wc, bcr)
